# Initial kernel scaffold; baseline (speedup 1.0000x reference)
import jax
import jax.numpy as jnp
from jax.experimental import pallas as pl


def kernel(nodes__author, nodes__paper, edges__author__writes__paper, edges__paper__written_by__author, Wk, Wq, Wv, Wo, bk, bq, bv, bo, a_rel, m_rel, p_rel, skip, ln_gamma, ln_beta):
    raise NotImplementedError("write your pallas kernel here")


# speedup vs baseline: 1.0372x; 1.0372x over previous
import functools

import jax
import jax.numpy as jnp
from jax.experimental import pallas as pl
from jax.experimental.pallas import tpu as pltpu

L = 2; T = 2; R = 2; H = 8; D = 256; DH = 32; N = 10000; E = 100000
BN = 1000
NP = 10016
HC = H // 2
DC = HC * DH
NUMW = 144

_INTERPRET = False


def _proj_body(x_ref, w_ref, b_ref, q_ref, kv_ref):
    y = jnp.dot(x_ref[...], w_ref[...], preferred_element_type=jnp.float32)
    y = y + b_ref[...]
    q_ref[0] = y[:, 0:DC]
    q_ref[1] = y[:, DC:2 * DC]
    kv_ref[0] = jnp.concatenate([y[:, 256:256 + DC], y[:, 512:512 + DC]], axis=1)
    kv_ref[1] = jnp.concatenate([y[:, 256 + DC:512], y[:, 512 + DC:768]], axis=1)


def _proj(x, wcat, bcat):
    grid = (N // BN,)
    return pl.pallas_call(
        _proj_body,
        grid=grid,
        in_specs=[
            pl.BlockSpec((BN, D), lambda i: (i, 0)),
            pl.BlockSpec((D, 3 * D), lambda i: (0, 0)),
            pl.BlockSpec((1, 3 * D), lambda i: (0, 0)),
        ],
        out_specs=[
            pl.BlockSpec((2, BN, DC), lambda i: (0, i, 0)),
            pl.BlockSpec((2, BN, 2 * DC), lambda i: (0, i, 0)),
        ],
        out_shape=[
            jax.ShapeDtypeStruct((2, N, DC), jnp.float32),
            jax.ShapeDtypeStruct((2, N, 2 * DC), jnp.float32),
        ],
        interpret=_INTERPRET,
    )(x, wcat, bcat)


def _post_body(num_ref, x_ref, wo_ref, bo_ref, beta_ref, g_ref, b2_ref, o_ref):
    a = jnp.concatenate([num_ref[0, :, 0:DC], num_ref[1, :, 0:DC]], axis=1)
    den = jnp.concatenate([num_ref[0, :, DC:DC + HC], num_ref[1, :, DC:DC + HC]],
                          axis=1)
    dfull = jnp.broadcast_to(den[:, :, None], (BN, H, DH)).reshape(BN, D)
    agg = a / (dfull + 1e-16)
    g = jax.nn.gelu(agg)
    o = jnp.dot(g, wo_ref[...], preferred_element_type=jnp.float32) + bo_ref[...]
    beta = beta_ref[0, 0]
    x = x_ref[...]
    y = x + beta * o + (1.0 - beta) * x
    mu = jnp.mean(y, axis=1, keepdims=True)
    var = jnp.mean((y - mu) ** 2, axis=1, keepdims=True)
    o_ref[...] = (y - mu) * jax.lax.rsqrt(var + 1e-5) * g_ref[...] + b2_ref[...]


def _post(num, x, wo, bo, beta, gamma, beta_ln):
    grid = (N // BN,)
    return pl.pallas_call(
        _post_body,
        grid=grid,
        in_specs=[
            pl.BlockSpec((2, BN, NUMW), lambda i: (0, i, 0)),
            pl.BlockSpec((BN, D), lambda i: (i, 0)),
            pl.BlockSpec((D, D), lambda i: (0, 0)),
            pl.BlockSpec((1, D), lambda i: (0, 0)),
            pl.BlockSpec((1, 1), lambda i: (0, 0), memory_space=pltpu.SMEM),
            pl.BlockSpec((1, D), lambda i: (0, 0)),
            pl.BlockSpec((1, D), lambda i: (0, 0)),
        ],
        out_specs=pl.BlockSpec((BN, D), lambda i: (i, 0)),
        out_shape=jax.ShapeDtypeStruct((N, D), jnp.float32),
        interpret=_INTERPRET,
    )(num, x, wo, bo, beta, gamma, beta_ln)


def _edge_pass(q, kv, src, dst):
    qf = jnp.concatenate([q[0], q[1]], axis=1).reshape(N, H, DH)
    kf = jnp.concatenate([kv[0][:, :DC], kv[1][:, :DC]], axis=1).reshape(N, H, DH)
    vf = jnp.concatenate([kv[0][:, DC:], kv[1][:, DC:]], axis=1).reshape(N, H, DH)
    sc = (qf[dst] * kf[src]).sum(-1)
    ex = jnp.exp(sc)
    den = jax.ops.segment_sum(ex, dst, num_segments=N)
    num = jax.ops.segment_sum(ex[:, :, None] * vf[src], dst, num_segments=N)
    numf = num.reshape(N, D)
    out = jnp.zeros((2, NP, NUMW), jnp.float32)
    for c in (0, 1):
        out = out.at[c, :N, :DC].set(numf[:, c * DC:(c + 1) * DC])
        out = out.at[c, :N, DC:DC + HC].set(den[:, c * HC:(c + 1) * HC])
    return out


def kernel(nodes__author, nodes__paper, edges__author__writes__paper,
           edges__paper__written_by__author, Wk, Wq, Wv, Wo, bk, bq, bv, bo,
           a_rel, m_rel, p_rel, skip, ln_gamma, ln_beta):
    scale = 1.0 / jnp.sqrt(jnp.float32(DH))
    a_s = a_rel * (p_rel * scale)[..., None, None]

    Wk_h = Wk.reshape(L, T, D, H, DH)
    Wv_h = Wv.reshape(L, T, D, H, DH)
    bk_h = bk.reshape(L, T, H, DH)
    bv_h = bv.reshape(L, T, H, DH)
    Kf = jnp.einsum('lrdhe,lrhef->lrdhf',
                    jnp.stack([Wk_h[:, 0], Wk_h[:, 1]], axis=1), a_s).reshape(L, R, D, D)
    Vf = jnp.einsum('lrdhe,lrhef->lrdhf',
                    jnp.stack([Wv_h[:, 0], Wv_h[:, 1]], axis=1), m_rel).reshape(L, R, D, D)
    bKf = jnp.einsum('lrhe,lrhef->lrhf',
                     jnp.stack([bk_h[:, 0], bk_h[:, 1]], axis=1), a_s).reshape(L, R, D)
    bVf = jnp.einsum('lrhe,lrhef->lrhf',
                     jnp.stack([bv_h[:, 0], bv_h[:, 1]], axis=1), m_rel).reshape(L, R, D)
    Wcat = jnp.concatenate([Wq, Kf, Vf], axis=3)
    bcat = jnp.concatenate([bq, bKf, bVf], axis=2)[:, :, None, :]

    betas = jax.nn.sigmoid(skip)

    e_ap = edges__author__writes__paper
    e_pa = edges__paper__written_by__author
    src_r = (e_ap[0].astype(jnp.int32), e_pa[0].astype(jnp.int32))
    dst_r = (e_ap[1].astype(jnp.int32), e_pa[1].astype(jnp.int32))

    x = [nodes__author, nodes__paper]
    for l in range(L):
        q = [None, None]
        kv = [None, None]
        for t in range(T):
            q[t], kv[t] = _proj(x[t], Wcat[l, t], bcat[l, t])
        num = [None, None]
        for r in range(R):
            num[r] = _edge_pass(q[1 - r], kv[r], src_r[r], dst_r[r])
        newx = []
        for t in range(T):
            r = 1 - t
            newx.append(_post(num[r], x[t], Wo[l, t], bo[l, t][None, :],
                              betas[l, t][None, None], ln_gamma[l][None, :],
                              ln_beta[l][None, :]))
        x = newx
    return jnp.stack(x, 0)


# speedup vs baseline: 8.6536x; 8.3429x over previous
import dataclasses
import functools

import jax
import jax.numpy as jnp
from jax import lax
from jax.experimental import pallas as pl
from jax.experimental.pallas import tpu as pltpu
from jax.experimental.pallas import tpu_sc as plsc

L = 2; T = 2; R = 2; H = 8; D = 256; DH = 32; N = 10000; E = 100000
BN = 1000
NH = 5000
NPH = 5120
NOUT = NH + NPH
NPD = 320
HC = H // 2
DC = HC * DH
NSUB = 16
EB = 128
NBLK = 49
EP = NSUB * EB * NBLK
RPN = NPH // NSUB

_INTERPRET = False


def _proj_body(x_ref, w_ref, b_ref, q_ref, k_ref, v_ref):
    y = jnp.dot(x_ref[...], w_ref[...], preferred_element_type=jnp.float32)
    y = y + b_ref[...]
    q_ref[0] = y[:, 0:DC]
    q_ref[1] = y[:, DC:2 * DC]
    k_ref[0] = y[:, 256:256 + DC]
    k_ref[1] = y[:, 256 + DC:512]
    v_ref[0] = y[:, 512:512 + DC]
    v_ref[1] = y[:, 512 + DC:768]


def _proj(x, wcat, bcat):
    out3 = [
        jax.ShapeDtypeStruct((2, N, DC), jnp.float32),
        jax.ShapeDtypeStruct((2, N, DC), jnp.float32),
        jax.ShapeDtypeStruct((2, N, DC), jnp.float32),
    ]
    return pl.pallas_call(
        _proj_body,
        grid=(N // BN,),
        in_specs=[
            pl.BlockSpec((BN, D), lambda i: (i, 0)),
            pl.BlockSpec((D, 3 * D), lambda i: (0, 0)),
            pl.BlockSpec((1, 3 * D), lambda i: (0, 0)),
        ],
        out_specs=[pl.BlockSpec((2, BN, DC), lambda i: (0, i, 0))] * 3,
        out_shape=out3,
        interpret=_INTERPRET,
    )(x, wcat, bcat)


def _post_body(num_ref, den_ref, x_ref, wo_ref, bo_ref, beta_ref, g_ref,
               b2_ref, o_ref):
    a = jnp.concatenate([num_ref[0], num_ref[1]], axis=1)
    den = den_ref[...]
    dfull = jnp.broadcast_to(den[:, :, None], (BN, H, DH)).reshape(BN, D)
    agg = a / (dfull + 1e-16)
    g = jax.nn.gelu(agg)
    o = jnp.dot(g, wo_ref[...], preferred_element_type=jnp.float32) + bo_ref[...]
    beta = beta_ref[0, 0]
    x = x_ref[...]
    y = x + beta * o + (1.0 - beta) * x
    mu = jnp.mean(y, axis=1, keepdims=True)
    var = jnp.mean((y - mu) ** 2, axis=1, keepdims=True)
    o_ref[...] = (y - mu) * jax.lax.rsqrt(var + 1e-5) * g_ref[...] + b2_ref[...]


def _post(num, den, x, wo, bo, beta, gamma, beta_ln):
    return pl.pallas_call(
        _post_body,
        grid=(N // BN,),
        in_specs=[
            pl.BlockSpec((2, BN, DC), lambda i: (0, i, 0)),
            pl.BlockSpec((BN, H), lambda i: (i, 0)),
            pl.BlockSpec((BN, D), lambda i: (i, 0)),
            pl.BlockSpec((D, D), lambda i: (0, 0)),
            pl.BlockSpec((1, D), lambda i: (0, 0)),
            pl.BlockSpec((1, 1), lambda i: (0, 0), memory_space=pltpu.SMEM),
            pl.BlockSpec((1, D), lambda i: (0, 0)),
            pl.BlockSpec((1, D), lambda i: (0, 0)),
        ],
        out_specs=pl.BlockSpec((BN, D), lambda i: (i, 0)),
        out_shape=jax.ShapeDtypeStruct((N, D), jnp.float32),
        interpret=_INTERPRET,
    )(num, den, x, wo, bo, beta, gamma, beta_ln)


def _zero_num(stgv, num_sh, s):
    rbase = RPN * s
    for i in range(RPN // EB):
        pltpu.sync_copy(stgv.at[pl.ds(0, EB)],
                        num_sh.at[pl.ds(rbase + i * EB, EB)])
    tail = RPN - (RPN // EB) * EB
    if tail:
        pltpu.sync_copy(stgv.at[pl.ds(0, tail)],
                        num_sh.at[pl.ds(rbase + (RPN // EB) * EB, tail)])


def _zero_stgv(stgv):
    zero16 = jnp.zeros((16,), jnp.float32)

    @pl.loop(0, EB)
    def _(row):
        for j in range(DC // 16):
            stgv[row, pl.ds(16 * j, 16)] = zero16


def _edge_one_relation(q_hbm, k_hbm, v_hbm, src_hbm, dst_hbm, outn_hbm,
                       outd_hbm, exc_hbm, srci, dsti, dstq, dstd, dstc, dstn,
                       qb, kb, vb, stgv, stgd, exb, num_sh, den_sh, c, s):
    zero16 = jnp.zeros((16,), jnp.float32)
    lane = lax.iota(jnp.int32, 16)
    lane8 = lax.shift_right_logical(lane, 3)
    lanec = jnp.bitwise_and(lane, 7) * 16

    _zero_stgv(stgv)

    @pl.loop(0, EB)
    def _(row):
        for j in range(DC // 16):
            stgd[row, pl.ds(16 * j, 16)] = zero16

    _zero_num(stgv, num_sh, s)

    @pl.when(s == 0)
    def _():
        pltpu.sync_copy(stgv.at[pl.ds(0, EB)], den_sh.at[pl.ds(0, EB)])
        pltpu.sync_copy(stgv.at[pl.ds(0, EB)], den_sh.at[pl.ds(EB, EB)])
        pltpu.sync_copy(stgv.at[pl.ds(0, NPD - 2 * EB)],
                        den_sh.at[pl.ds(2 * EB, NPD - 2 * EB)])
    plsc.subcore_barrier()

    cN = c * N
    rbase = RPN * s

    @pl.loop(0, NBLK)
    def _(i):
        ebase = (s * NBLK + i) * EB
        pltpu.sync_copy(src_hbm.at[pl.ds(ebase, EB)], srci)
        pltpu.sync_copy(dst_hbm.at[pl.ds(ebase, EB)], dsti)

        @pl.loop(0, EB // 16)
        def _(j):
            sl = pl.ds(16 * j, 16)
            dv = dsti[sl]
            srci[sl] = srci[sl] + cN
            dstq[sl] = jnp.minimum(dv, N - 1) + cN
            dstd[sl] = lax.shift_right_logical(dv, 5)
            dstc[sl] = jnp.bitwise_and(dv, 31) * 4
            dstn[sl] = jnp.minimum(dv, NPH - 1)

        pltpu.sync_copy(q_hbm.at[dstq], qb)
        pltpu.sync_copy(k_hbm.at[srci], kb)
        pltpu.sync_copy(v_hbm.at[srci], vb)

        @pl.loop(0, EB)
        def _(e):
            denv = zero16
            for h in range(HC):
                a = qb[e, pl.ds(32 * h, 16)] * kb[e, pl.ds(32 * h, 16)]
                b = qb[e, pl.ds(32 * h + 16, 16)] * kb[e, pl.ds(32 * h + 16, 16)]
                tot = jnp.sum(a + b)
                exv = jnp.exp(jnp.broadcast_to(tot, (16,)))
                stgv[e, pl.ds(32 * h, 16)] = vb[e, pl.ds(32 * h, 16)] * exv
                stgv[e, pl.ds(32 * h + 16, 16)] = \
                    vb[e, pl.ds(32 * h + 16, 16)] * exv
                denv = jnp.where(lane == h, exv, denv)
            exb[lax.shift_right_logical(e, 3),
                pl.ds(jnp.bitwise_and(e, 7) * 16, 16)] = denv

        pltpu.sync_copy(exb, exc_hbm.at[c, pl.ds((s * NBLK + i) * (EB // 8),
                                                 EB // 8)])

        @pl.loop(0, EB // 16)
        def _(j):
            rows = jnp.broadcast_to(2 * j, (16,)) + lane8
            erow = jnp.broadcast_to(16 * j, (16,)) + lane
            cv = dstc[pl.ds(16 * j, 16)]
            for h in range(HC):
                exv = plsc.load_gather(exb, [rows, lanec + h])
                plsc.store_scatter(stgd, [erow, cv + h], exv)

        pltpu.sync_copy(stgv, num_sh.at[dstn], add=True)
        pltpu.sync_copy(stgd, den_sh.at[dstd], add=True)

        @pl.loop(0, EB // 16)
        def _(j):
            erow = jnp.broadcast_to(16 * j, (16,)) + lane
            cv = dstc[pl.ds(16 * j, 16)]
            for h in range(HC):
                plsc.store_scatter(stgd, [erow, cv + h], zero16)

    plsc.subcore_barrier()
    pltpu.sync_copy(num_sh.at[pl.ds(rbase, RPN)],
                    outn_hbm.at[c, pl.ds(rbase, RPN)])

    @pl.when(s < 4)
    def _():
        dbase = s * (NPD // 4)
        pltpu.sync_copy(den_sh.at[pl.ds(dbase, NPD // 4)],
                        outd_hbm.at[c, pl.ds(dbase, NPD // 4)])
    plsc.subcore_barrier()

    _zero_stgv(stgv)
    _zero_num(stgv, num_sh, s)
    plsc.subcore_barrier()

    @pl.loop(0, NBLK)
    def _(i):
        ebase = (s * NBLK + i) * EB
        pltpu.sync_copy(src_hbm.at[pl.ds(ebase, EB)], srci)
        pltpu.sync_copy(dst_hbm.at[pl.ds(ebase, EB)], dsti)

        @pl.loop(0, EB // 16)
        def _(j):
            sl = pl.ds(16 * j, 16)
            dv = dsti[sl]
            srci[sl] = srci[sl] + cN
            dstn[sl] = jnp.where(dv >= NH, dv - NH, NPH - 1)

        pltpu.sync_copy(v_hbm.at[srci], vb)
        pltpu.sync_copy(exc_hbm.at[c, pl.ds((s * NBLK + i) * (EB // 8),
                                            EB // 8)], exb)

        @pl.loop(0, EB)
        def _(e):
            er = lax.shift_right_logical(e, 3)
            ec = jnp.bitwise_and(e, 7) * 16
            for h in range(HC):
                exv = plsc.load_gather(
                    exb, [jnp.broadcast_to(er, (16,)),
                          jnp.broadcast_to(ec + h, (16,))])
                stgv[e, pl.ds(32 * h, 16)] = vb[e, pl.ds(32 * h, 16)] * exv
                stgv[e, pl.ds(32 * h + 16, 16)] = \
                    vb[e, pl.ds(32 * h + 16, 16)] * exv

        pltpu.sync_copy(stgv, num_sh.at[dstn], add=True)

    plsc.subcore_barrier()
    pltpu.sync_copy(num_sh.at[pl.ds(rbase, RPN)],
                    outn_hbm.at[c, pl.ds(NH + rbase, RPN)])
    plsc.subcore_barrier()


def _edge_sc_body(q0, k0, v0, src0, dst0, q1, k1, v1, src1, dst1,
                  outn0, outd0, exc0, outn1, outd1, exc1,
                  srci, dsti, dstq, dstd, dstc, dstn, qb, kb, vb, stgv, stgd,
                  exb, num_sh, den_sh):
    c = lax.axis_index("c")
    s = lax.axis_index("s")
    _edge_one_relation(q0, k0, v0, src0, dst0, outn0, outd0, exc0, srci, dsti,
                       dstq, dstd, dstc, dstn, qb, kb, vb, stgv, stgd, exb,
                       num_sh, den_sh, c, s)
    _edge_one_relation(q1, k1, v1, src1, dst1, outn1, outd1, exc1, srci, dsti,
                       dstq, dstd, dstc, dstn, qb, kb, vb, stgv, stgd, exb,
                       num_sh, den_sh, c, s)


_sc_params = pltpu.CompilerParams()
if "needs_layout_passes" in pltpu.CompilerParams.__dataclass_fields__:
    _sc_params = dataclasses.replace(_sc_params, needs_layout_passes=False)

_edge_sc = functools.partial(
    pl.kernel,
    out_type=[
        jax.ShapeDtypeStruct((2, NOUT, DC), jnp.float32),
        jax.ShapeDtypeStruct((2, NPD, DC), jnp.float32),
        jax.ShapeDtypeStruct((2, EP // 8, DC), jnp.float32),
        jax.ShapeDtypeStruct((2, NOUT, DC), jnp.float32),
        jax.ShapeDtypeStruct((2, NPD, DC), jnp.float32),
        jax.ShapeDtypeStruct((2, EP // 8, DC), jnp.float32),
    ],
    mesh=plsc.VectorSubcoreMesh(core_axis_name="c", subcore_axis_name="s"),
    compiler_params=_sc_params,
    scratch_types=[
        pltpu.VMEM((EB,), jnp.int32),
        pltpu.VMEM((EB,), jnp.int32),
        pltpu.VMEM((EB,), jnp.int32),
        pltpu.VMEM((EB,), jnp.int32),
        pltpu.VMEM((EB,), jnp.int32),
        pltpu.VMEM((EB,), jnp.int32),
        pltpu.VMEM((EB, DC), jnp.float32),
        pltpu.VMEM((EB, DC), jnp.float32),
        pltpu.VMEM((EB, DC), jnp.float32),
        pltpu.VMEM((EB, DC), jnp.float32),
        pltpu.VMEM((EB, DC), jnp.float32),
        pltpu.VMEM((EB // 8, DC), jnp.float32),
        pltpu.VMEM_SHARED((NPH, DC), jnp.float32),
        pltpu.VMEM_SHARED((NPD, DC), jnp.float32),
    ],
)(_edge_sc_body)


def kernel(nodes__author, nodes__paper, edges__author__writes__paper,
           edges__paper__written_by__author, Wk, Wq, Wv, Wo, bk, bq, bv, bo,
           a_rel, m_rel, p_rel, skip, ln_gamma, ln_beta):
    scale = 1.0 / jnp.sqrt(jnp.float32(DH))
    a_s = a_rel * (p_rel * scale)[..., None, None]

    Wk_h = Wk.reshape(L, T, D, H, DH)
    Wv_h = Wv.reshape(L, T, D, H, DH)
    bk_h = bk.reshape(L, T, H, DH)
    bv_h = bv.reshape(L, T, H, DH)
    Kf = jnp.einsum('lrdhe,lrhef->lrdhf',
                    jnp.stack([Wk_h[:, 0], Wk_h[:, 1]], axis=1), a_s).reshape(L, R, D, D)
    Vf = jnp.einsum('lrdhe,lrhef->lrdhf',
                    jnp.stack([Wv_h[:, 0], Wv_h[:, 1]], axis=1), m_rel).reshape(L, R, D, D)
    bKf = jnp.einsum('lrhe,lrhef->lrhf',
                     jnp.stack([bk_h[:, 0], bk_h[:, 1]], axis=1), a_s).reshape(L, R, D)
    bVf = jnp.einsum('lrhe,lrhef->lrhf',
                     jnp.stack([bv_h[:, 0], bv_h[:, 1]], axis=1), m_rel).reshape(L, R, D)
    Wcat = jnp.concatenate([Wq, Kf, Vf], axis=3)
    bcat = jnp.concatenate([bq, bKf, bVf], axis=2)[:, :, None, :]

    betas = jax.nn.sigmoid(skip)

    def _pad_edges(e):
        src = jnp.concatenate([e[0].astype(jnp.int32),
                               jnp.zeros((EP - E,), jnp.int32)])
        dst = jnp.concatenate([e[1].astype(jnp.int32),
                               jnp.full((EP - E,), N, jnp.int32)])
        return src, dst

    src_r, dst_r = zip(_pad_edges(edges__author__writes__paper),
                       _pad_edges(edges__paper__written_by__author))

    def layer_body(carry, wl):
        x = list(carry)
        Wcat_l, bcat_l, Wo_l, bo_l, betas_l, gamma_l, bln_l = wl
        q = [None, None]; k = [None, None]; v = [None, None]
        for t in range(T):
            qt, kt, vt = _proj(x[t], Wcat_l[t], bcat_l[t])
            q[t] = qt.reshape(2 * N, DC)
            k[t] = kt.reshape(2 * N, DC)
            v[t] = vt.reshape(2 * N, DC)
        numn0, dend0, _exc0, numn1, dend1, _exc1 = _edge_sc(
            q[1], k[0], v[0], src_r[0], dst_r[0],
            q[0], k[1], v[1], src_r[1], dst_r[1])
        nums = [numn0, numn1]
        dens = []
        for r in range(R):
            d8 = [dend0, dend1][r].reshape(2, NPD * 32, 4)[:, :N, :]
            dens.append(jnp.transpose(d8, (1, 0, 2)).reshape(N, H))
        newx = []
        for t in range(T):
            r = 1 - t
            newx.append(_post(nums[r], dens[r], x[t], Wo_l[t],
                              bo_l[t][None, :], betas_l[t][None, None],
                              gamma_l[None, :], bln_l[None, :]))
        return tuple(newx), None

    carry, _ = lax.scan(layer_body, (nodes__author, nodes__paper),
                        (Wcat, bcat, Wo, bo, betas, ln_gamma, ln_beta))
    return jnp.stack(list(carry), 0)


# speedup vs baseline: 17.1931x; 1.9868x over previous
import dataclasses
import functools

import jax
import jax.numpy as jnp
from jax import lax
from jax.experimental import pallas as pl
from jax.experimental.pallas import tpu as pltpu
from jax.experimental.pallas import tpu_sc as plsc

L = 2; T = 2; R = 2; H = 8; D = 256; DH = 32; N = 10000; E = 100000
BN = 1000
NH = 5000
NPH = 5120
NOUT = NH + NPH
NPD = 320
HC = H // 2
DC = HC * DH
NSUB = 16
EB = 128
NBLK = 49
EP = NSUB * EB * NBLK
RPN = NPH // NSUB

_INTERPRET = False


def _proj_body(x_ref, w_ref, b_ref, q_ref, k_ref, v_ref):
    y = jnp.dot(x_ref[...], w_ref[...], preferred_element_type=jnp.float32)
    y = y + b_ref[...]
    q_ref[0] = y[:, 0:DC]
    q_ref[1] = y[:, DC:2 * DC]
    k_ref[0] = y[:, 256:256 + DC]
    k_ref[1] = y[:, 256 + DC:512]
    v_ref[0] = y[:, 512:512 + DC]
    v_ref[1] = y[:, 512 + DC:768]


def _proj(x, wcat, bcat):
    out3 = [
        jax.ShapeDtypeStruct((2, N, DC), jnp.float32),
        jax.ShapeDtypeStruct((2, N, DC), jnp.float32),
        jax.ShapeDtypeStruct((2, N, DC), jnp.float32),
    ]
    return pl.pallas_call(
        _proj_body,
        grid=(N // BN,),
        in_specs=[
            pl.BlockSpec((BN, D), lambda i: (i, 0)),
            pl.BlockSpec((D, 3 * D), lambda i: (0, 0)),
            pl.BlockSpec((1, 3 * D), lambda i: (0, 0)),
        ],
        out_specs=[pl.BlockSpec((2, BN, DC), lambda i: (0, i, 0))] * 3,
        out_shape=out3,
        interpret=_INTERPRET,
    )(x, wcat, bcat)


def _post_body(num_ref, den_ref, x_ref, wo_ref, bo_ref, beta_ref, g_ref,
               b2_ref, o_ref):
    a = jnp.concatenate([num_ref[0], num_ref[1]], axis=1)
    den = den_ref[...]
    dfull = jnp.broadcast_to(den[:, :, None], (BN, H, DH)).reshape(BN, D)
    agg = a / (dfull + 1e-16)
    g = jax.nn.gelu(agg)
    o = jnp.dot(g, wo_ref[...], preferred_element_type=jnp.float32) + bo_ref[...]
    beta = beta_ref[0, 0]
    x = x_ref[...]
    y = x + beta * o + (1.0 - beta) * x
    mu = jnp.mean(y, axis=1, keepdims=True)
    var = jnp.mean((y - mu) ** 2, axis=1, keepdims=True)
    o_ref[...] = (y - mu) * jax.lax.rsqrt(var + 1e-5) * g_ref[...] + b2_ref[...]


def _post(num, den, x, wo, bo, beta, gamma, beta_ln):
    return pl.pallas_call(
        _post_body,
        grid=(N // BN,),
        in_specs=[
            pl.BlockSpec((2, BN, DC), lambda i: (0, i, 0)),
            pl.BlockSpec((BN, H), lambda i: (i, 0)),
            pl.BlockSpec((BN, D), lambda i: (i, 0)),
            pl.BlockSpec((D, D), lambda i: (0, 0)),
            pl.BlockSpec((1, D), lambda i: (0, 0)),
            pl.BlockSpec((1, 1), lambda i: (0, 0), memory_space=pltpu.SMEM),
            pl.BlockSpec((1, D), lambda i: (0, 0)),
            pl.BlockSpec((1, D), lambda i: (0, 0)),
        ],
        out_specs=pl.BlockSpec((BN, D), lambda i: (i, 0)),
        out_shape=jax.ShapeDtypeStruct((N, D), jnp.float32),
        interpret=_INTERPRET,
    )(num, den, x, wo, bo, beta, gamma, beta_ln)


def _zero_num(stgv, num_sh, s):
    rbase = RPN * s
    for i in range(RPN // EB):
        pltpu.sync_copy(stgv.at[pl.ds(0, EB)],
                        num_sh.at[pl.ds(rbase + i * EB, EB)])
    tail = RPN - (RPN // EB) * EB
    if tail:
        pltpu.sync_copy(stgv.at[pl.ds(0, tail)],
                        num_sh.at[pl.ds(rbase + (RPN // EB) * EB, tail)])


def _zero_stgv(stgv):
    zero16 = jnp.zeros((16,), jnp.float32)

    @pl.loop(0, EB)
    def _(row):
        for j in range(DC // 16):
            stgv[row, pl.ds(16 * j, 16)] = zero16


def _edge_one_relation(q_hbm, k_hbm, v_hbm, src_hbm, dst_hbm, outn_hbm,
                       outd_hbm, exc_hbm, srci, dsti, dstq, dstd, dstc, dstn,
                       qb, kb, vb, stgv, stgd, exb, num_sh, den_sh, c, s):
    zero16 = jnp.zeros((16,), jnp.float32)
    lane = lax.iota(jnp.int32, 16)
    lane8 = lax.shift_right_logical(lane, 3)
    lanec = jnp.bitwise_and(lane, 7) * 16

    _zero_stgv(stgv)

    @pl.loop(0, EB)
    def _(row):
        for j in range(DC // 16):
            stgd[row, pl.ds(16 * j, 16)] = zero16

    _zero_num(stgv, num_sh, s)

    @pl.when(s == 0)
    def _():
        pltpu.sync_copy(stgv.at[pl.ds(0, EB)], den_sh.at[pl.ds(0, EB)])
        pltpu.sync_copy(stgv.at[pl.ds(0, EB)], den_sh.at[pl.ds(EB, EB)])
        pltpu.sync_copy(stgv.at[pl.ds(0, NPD - 2 * EB)],
                        den_sh.at[pl.ds(2 * EB, NPD - 2 * EB)])
    plsc.subcore_barrier()

    cN = c * N
    rbase = RPN * s

    @pl.loop(0, NBLK)
    def _(i):
        ebase = (s * NBLK + i) * EB
        pltpu.sync_copy(src_hbm.at[pl.ds(ebase, EB)], srci)
        pltpu.sync_copy(dst_hbm.at[pl.ds(ebase, EB)], dsti)

        @pl.loop(0, EB // 16)
        def _(j):
            sl = pl.ds(16 * j, 16)
            dv = dsti[sl]
            srci[sl] = srci[sl] + cN
            dstq[sl] = jnp.minimum(dv, N - 1) + cN
            dstd[sl] = lax.shift_right_logical(dv, 5)
            dstc[sl] = jnp.bitwise_and(dv, 31) * 4
            dstn[sl] = jnp.minimum(dv, NPH - 1)

        pltpu.sync_copy(q_hbm.at[dstq], qb)
        pltpu.sync_copy(k_hbm.at[srci], kb)
        pltpu.sync_copy(v_hbm.at[srci], vb)

        @plsc.parallel_loop(0, EB, unroll=2)
        def _(e):
            ps = []
            for h in range(HC):
                a = qb[e, pl.ds(32 * h, 16)] * kb[e, pl.ds(32 * h, 16)]
                b = qb[e, pl.ds(32 * h + 16, 16)] * kb[e, pl.ds(32 * h + 16, 16)]
                ps.append(a + b)
            tots = [jnp.sum(p) for p in ps]
            exvs = [jnp.exp(jnp.broadcast_to(t, (16,))) for t in tots]
            denv = zero16
            for h in range(HC):
                stgv[e, pl.ds(32 * h, 16)] = vb[e, pl.ds(32 * h, 16)] * exvs[h]
                stgv[e, pl.ds(32 * h + 16, 16)] = \
                    vb[e, pl.ds(32 * h + 16, 16)] * exvs[h]
                denv = jnp.where(lane == h, exvs[h], denv)
            exb[lax.shift_right_logical(e, 3),
                pl.ds(jnp.bitwise_and(e, 7) * 16, 16)] = denv

        pltpu.sync_copy(exb, exc_hbm.at[c, pl.ds((s * NBLK + i) * (EB // 8),
                                                 EB // 8)])

        @pl.loop(0, EB // 16)
        def _(j):
            rows = jnp.broadcast_to(2 * j, (16,)) + lane8
            erow = jnp.broadcast_to(16 * j, (16,)) + lane
            cv = dstc[pl.ds(16 * j, 16)]
            for h in range(HC):
                exv = plsc.load_gather(exb, [rows, lanec + h])
                plsc.store_scatter(stgd, [erow, cv + h], exv)

        pltpu.sync_copy(stgv, num_sh.at[dstn], add=True)
        pltpu.sync_copy(stgd, den_sh.at[dstd], add=True)

        @pl.loop(0, EB // 16)
        def _(j):
            erow = jnp.broadcast_to(16 * j, (16,)) + lane
            cv = dstc[pl.ds(16 * j, 16)]
            for h in range(HC):
                plsc.store_scatter(stgd, [erow, cv + h], zero16)

    plsc.subcore_barrier()
    pltpu.sync_copy(num_sh.at[pl.ds(rbase, RPN)],
                    outn_hbm.at[c, pl.ds(rbase, RPN)])

    @pl.when(s < 4)
    def _():
        dbase = s * (NPD // 4)
        pltpu.sync_copy(den_sh.at[pl.ds(dbase, NPD // 4)],
                        outd_hbm.at[c, pl.ds(dbase, NPD // 4)])
    plsc.subcore_barrier()

    _zero_stgv(stgv)
    _zero_num(stgv, num_sh, s)
    plsc.subcore_barrier()

    @pl.loop(0, NBLK)
    def _(i):
        ebase = (s * NBLK + i) * EB
        pltpu.sync_copy(src_hbm.at[pl.ds(ebase, EB)], srci)
        pltpu.sync_copy(dst_hbm.at[pl.ds(ebase, EB)], dsti)

        @pl.loop(0, EB // 16)
        def _(j):
            sl = pl.ds(16 * j, 16)
            dv = dsti[sl]
            srci[sl] = srci[sl] + cN
            dstn[sl] = jnp.where(dv >= NH, dv - NH, NPH - 1)

        pltpu.sync_copy(v_hbm.at[srci], vb)
        pltpu.sync_copy(exc_hbm.at[c, pl.ds((s * NBLK + i) * (EB // 8),
                                            EB // 8)], exb)

        @plsc.parallel_loop(0, EB, unroll=4)
        def _(e):
            er = lax.shift_right_logical(e, 3)
            ec = jnp.bitwise_and(e, 7) * 16
            exvs = [plsc.load_gather(
                exb, [jnp.broadcast_to(er, (16,)),
                      jnp.broadcast_to(ec + h, (16,))]) for h in range(HC)]
            for h in range(HC):
                stgv[e, pl.ds(32 * h, 16)] = vb[e, pl.ds(32 * h, 16)] * exvs[h]
                stgv[e, pl.ds(32 * h + 16, 16)] = \
                    vb[e, pl.ds(32 * h + 16, 16)] * exvs[h]

        pltpu.sync_copy(stgv, num_sh.at[dstn], add=True)

    plsc.subcore_barrier()
    pltpu.sync_copy(num_sh.at[pl.ds(rbase, RPN)],
                    outn_hbm.at[c, pl.ds(NH + rbase, RPN)])
    plsc.subcore_barrier()


def _edge_sc_body(q0, k0, v0, src0, dst0, q1, k1, v1, src1, dst1,
                  outn0, outd0, exc0, outn1, outd1, exc1,
                  srci, dsti, dstq, dstd, dstc, dstn, qb, kb, vb, stgv, stgd,
                  exb, num_sh, den_sh):
    c = lax.axis_index("c")
    s = lax.axis_index("s")
    _edge_one_relation(q0, k0, v0, src0, dst0, outn0, outd0, exc0, srci, dsti,
                       dstq, dstd, dstc, dstn, qb, kb, vb, stgv, stgd, exb,
                       num_sh, den_sh, c, s)
    _edge_one_relation(q1, k1, v1, src1, dst1, outn1, outd1, exc1, srci, dsti,
                       dstq, dstd, dstc, dstn, qb, kb, vb, stgv, stgd, exb,
                       num_sh, den_sh, c, s)


_sc_params = pltpu.CompilerParams()
if "needs_layout_passes" in pltpu.CompilerParams.__dataclass_fields__:
    _sc_params = dataclasses.replace(_sc_params, needs_layout_passes=False)

_edge_sc = functools.partial(
    pl.kernel,
    out_type=[
        jax.ShapeDtypeStruct((2, NOUT, DC), jnp.float32),
        jax.ShapeDtypeStruct((2, NPD, DC), jnp.float32),
        jax.ShapeDtypeStruct((2, EP // 8, DC), jnp.float32),
        jax.ShapeDtypeStruct((2, NOUT, DC), jnp.float32),
        jax.ShapeDtypeStruct((2, NPD, DC), jnp.float32),
        jax.ShapeDtypeStruct((2, EP // 8, DC), jnp.float32),
    ],
    mesh=plsc.VectorSubcoreMesh(core_axis_name="c", subcore_axis_name="s"),
    compiler_params=_sc_params,
    scratch_types=[
        pltpu.VMEM((EB,), jnp.int32),
        pltpu.VMEM((EB,), jnp.int32),
        pltpu.VMEM((EB,), jnp.int32),
        pltpu.VMEM((EB,), jnp.int32),
        pltpu.VMEM((EB,), jnp.int32),
        pltpu.VMEM((EB,), jnp.int32),
        pltpu.VMEM((EB, DC), jnp.float32),
        pltpu.VMEM((EB, DC), jnp.float32),
        pltpu.VMEM((EB, DC), jnp.float32),
        pltpu.VMEM((EB, DC), jnp.float32),
        pltpu.VMEM((EB, DC), jnp.float32),
        pltpu.VMEM((EB // 8, DC), jnp.float32),
        pltpu.VMEM_SHARED((NPH, DC), jnp.float32),
        pltpu.VMEM_SHARED((NPD, DC), jnp.float32),
    ],
)(_edge_sc_body)


def kernel(nodes__author, nodes__paper, edges__author__writes__paper,
           edges__paper__written_by__author, Wk, Wq, Wv, Wo, bk, bq, bv, bo,
           a_rel, m_rel, p_rel, skip, ln_gamma, ln_beta):
    scale = 1.0 / jnp.sqrt(jnp.float32(DH))
    a_s = a_rel * (p_rel * scale)[..., None, None]

    Wk_h = Wk.reshape(L, T, D, H, DH)
    Wv_h = Wv.reshape(L, T, D, H, DH)
    bk_h = bk.reshape(L, T, H, DH)
    bv_h = bv.reshape(L, T, H, DH)
    Kf = jnp.einsum('lrdhe,lrhef->lrdhf',
                    jnp.stack([Wk_h[:, 0], Wk_h[:, 1]], axis=1), a_s).reshape(L, R, D, D)
    Vf = jnp.einsum('lrdhe,lrhef->lrdhf',
                    jnp.stack([Wv_h[:, 0], Wv_h[:, 1]], axis=1), m_rel).reshape(L, R, D, D)
    bKf = jnp.einsum('lrhe,lrhef->lrhf',
                     jnp.stack([bk_h[:, 0], bk_h[:, 1]], axis=1), a_s).reshape(L, R, D)
    bVf = jnp.einsum('lrhe,lrhef->lrhf',
                     jnp.stack([bv_h[:, 0], bv_h[:, 1]], axis=1), m_rel).reshape(L, R, D)
    Wcat = jnp.concatenate([Wq, Kf, Vf], axis=3)
    bcat = jnp.concatenate([bq, bKf, bVf], axis=2)[:, :, None, :]

    betas = jax.nn.sigmoid(skip)

    def _pad_edges(e):
        src = jnp.concatenate([e[0].astype(jnp.int32),
                               jnp.zeros((EP - E,), jnp.int32)])
        dst = jnp.concatenate([e[1].astype(jnp.int32),
                               jnp.full((EP - E,), N, jnp.int32)])
        return src, dst

    src_r, dst_r = zip(_pad_edges(edges__author__writes__paper),
                       _pad_edges(edges__paper__written_by__author))

    def layer_body(carry, wl):
        x = list(carry)
        Wcat_l, bcat_l, Wo_l, bo_l, betas_l, gamma_l, bln_l = wl
        q = [None, None]; k = [None, None]; v = [None, None]
        for t in range(T):
            qt, kt, vt = _proj(x[t], Wcat_l[t], bcat_l[t])
            q[t] = qt.reshape(2 * N, DC)
            k[t] = kt.reshape(2 * N, DC)
            v[t] = vt.reshape(2 * N, DC)
        numn0, dend0, _exc0, numn1, dend1, _exc1 = _edge_sc(
            q[1], k[0], v[0], src_r[0], dst_r[0],
            q[0], k[1], v[1], src_r[1], dst_r[1])
        nums = [numn0, numn1]
        dens = []
        for r in range(R):
            d8 = [dend0, dend1][r].reshape(2, NPD * 32, 4)[:, :N, :]
            dens.append(jnp.transpose(d8, (1, 0, 2)).reshape(N, H))
        newx = []
        for t in range(T):
            r = 1 - t
            newx.append(_post(nums[r], dens[r], x[t], Wo_l[t],
                              bo_l[t][None, :], betas_l[t][None, None],
                              gamma_l[None, :], bln_l[None, :]))
        return tuple(newx), None

    carry, _ = lax.scan(layer_body, (nodes__author, nodes__paper),
                        (Wcat, bcat, Wo, bo, betas, ln_gamma, ln_beta))
    return jnp.stack(list(carry), 0)


# speedup vs baseline: 20.0210x; 1.1645x over previous
import dataclasses
import functools

import jax
import jax.numpy as jnp
from jax import lax
from jax.experimental import pallas as pl
from jax.experimental.pallas import tpu as pltpu
from jax.experimental.pallas import tpu_sc as plsc

L = 2; T = 2; R = 2; H = 8; D = 256; DH = 32; N = 10000; E = 100000
BN = 1000
NH = 5000
NPH = 5120
NOUT = NH + NPH
NPD = 320
HC = H // 2
DC = HC * DH
NSUB = 16
EB = 128
NBLK = 49
EP = NSUB * EB * NBLK
RPN = NPH // NSUB

_INTERPRET = False


def _proj_body(x_ref, w_ref, b_ref, q_ref, k_ref, v_ref):
    y = jnp.dot(x_ref[...], w_ref[...], preferred_element_type=jnp.float32)
    y = y + b_ref[...]
    q_ref[0] = y[:, 0:DC]
    q_ref[1] = y[:, DC:2 * DC]
    k_ref[0] = y[:, 256:256 + DC]
    k_ref[1] = y[:, 256 + DC:512]
    v_ref[0] = y[:, 512:512 + DC]
    v_ref[1] = y[:, 512 + DC:768]


def _proj(x, wcat, bcat):
    out3 = [
        jax.ShapeDtypeStruct((2, N, DC), jnp.float32),
        jax.ShapeDtypeStruct((2, N, DC), jnp.float32),
        jax.ShapeDtypeStruct((2, N, DC), jnp.float32),
    ]
    return pl.pallas_call(
        _proj_body,
        grid=(N // BN,),
        in_specs=[
            pl.BlockSpec((BN, D), lambda i: (i, 0)),
            pl.BlockSpec((D, 3 * D), lambda i: (0, 0)),
            pl.BlockSpec((1, 3 * D), lambda i: (0, 0)),
        ],
        out_specs=[pl.BlockSpec((2, BN, DC), lambda i: (0, i, 0))] * 3,
        out_shape=out3,
        interpret=_INTERPRET,
    )(x, wcat, bcat)


def _post_body(num_ref, den_ref, x_ref, wo_ref, bo_ref, beta_ref, g_ref,
               b2_ref, o_ref):
    a = jnp.concatenate([num_ref[0], num_ref[1]], axis=1)
    den = den_ref[...]
    dfull = jnp.broadcast_to(den[:, :, None], (BN, H, DH)).reshape(BN, D)
    agg = a / (dfull + 1e-16)
    g = jax.nn.gelu(agg)
    o = jnp.dot(g, wo_ref[...], preferred_element_type=jnp.float32) + bo_ref[...]
    beta = beta_ref[0, 0]
    x = x_ref[...]
    y = x + beta * o + (1.0 - beta) * x
    mu = jnp.mean(y, axis=1, keepdims=True)
    var = jnp.mean((y - mu) ** 2, axis=1, keepdims=True)
    o_ref[...] = (y - mu) * jax.lax.rsqrt(var + 1e-5) * g_ref[...] + b2_ref[...]


def _post(num, den, x, wo, bo, beta, gamma, beta_ln):
    return pl.pallas_call(
        _post_body,
        grid=(N // BN,),
        in_specs=[
            pl.BlockSpec((2, BN, DC), lambda i: (0, i, 0)),
            pl.BlockSpec((BN, H), lambda i: (i, 0)),
            pl.BlockSpec((BN, D), lambda i: (i, 0)),
            pl.BlockSpec((D, D), lambda i: (0, 0)),
            pl.BlockSpec((1, D), lambda i: (0, 0)),
            pl.BlockSpec((1, 1), lambda i: (0, 0), memory_space=pltpu.SMEM),
            pl.BlockSpec((1, D), lambda i: (0, 0)),
            pl.BlockSpec((1, D), lambda i: (0, 0)),
        ],
        out_specs=pl.BlockSpec((BN, D), lambda i: (i, 0)),
        out_shape=jax.ShapeDtypeStruct((N, D), jnp.float32),
        interpret=_INTERPRET,
    )(num, den, x, wo, bo, beta, gamma, beta_ln)


def _zero_num(stgv, num_sh, s):
    rbase = RPN * s
    for i in range(RPN // EB):
        pltpu.sync_copy(stgv.at[pl.ds(0, EB)],
                        num_sh.at[pl.ds(rbase + i * EB, EB)])
    tail = RPN - (RPN // EB) * EB
    if tail:
        pltpu.sync_copy(stgv.at[pl.ds(0, tail)],
                        num_sh.at[pl.ds(rbase + (RPN // EB) * EB, tail)])


def _zero_stgv(stgv):
    zero16 = jnp.zeros((16,), jnp.float32)

    @pl.loop(0, EB)
    def _(row):
        for j in range(DC // 16):
            stgv[row, pl.ds(16 * j, 16)] = zero16


def _edge_one_relation(q_hbm, k_hbm, v_hbm, src_hbm, dst_hbm, outn_hbm,
                       outd_hbm, exc_hbm, srci, dsti, dstq, dstd, dstc, dstn,
                       qb, kb, vb, stgv, stgd, exb, num_sh, den_sh, semg, sems,
                       c, s):
    zero16 = jnp.zeros((16,), jnp.float32)
    lane = lax.iota(jnp.int32, 16)
    lane8 = lax.shift_right_logical(lane, 3)
    lanec = jnp.bitwise_and(lane, 7) * 16

    _zero_stgv(stgv)

    @pl.loop(0, EB)
    def _(row):
        for j in range(DC // 16):
            stgd[row, pl.ds(16 * j, 16)] = zero16

    _zero_num(stgv, num_sh, s)

    @pl.when(s == 0)
    def _():
        pltpu.sync_copy(stgv.at[pl.ds(0, EB)], den_sh.at[pl.ds(0, EB)])
        pltpu.sync_copy(stgv.at[pl.ds(0, EB)], den_sh.at[pl.ds(EB, EB)])
        pltpu.sync_copy(stgv.at[pl.ds(0, NPD - 2 * EB)],
                        den_sh.at[pl.ds(2 * EB, NPD - 2 * EB)])
    plsc.subcore_barrier()

    cN = c * N
    rbase = RPN * s

    @pl.loop(0, NBLK)
    def _(i):
        ebase = (s * NBLK + i) * EB
        pltpu.sync_copy(src_hbm.at[pl.ds(ebase, EB)], srci)
        pltpu.sync_copy(dst_hbm.at[pl.ds(ebase, EB)], dsti)

        @pl.loop(0, EB // 16)
        def _(j):
            sl = pl.ds(16 * j, 16)
            dv = dsti[sl]
            srci[sl] = srci[sl] + cN
            dstq[sl] = jnp.minimum(dv, N - 1) + cN
            dstd[sl] = lax.shift_right_logical(dv, 5)
            dstc[sl] = jnp.bitwise_and(dv, 31) * 4
            dstn[sl] = jnp.minimum(dv, NPH - 1)

        cq = pltpu.async_copy(q_hbm.at[dstq], qb, semg)
        ck = pltpu.async_copy(k_hbm.at[srci], kb, semg)
        cv = pltpu.async_copy(v_hbm.at[srci], vb, semg)
        cq.wait(); ck.wait(); cv.wait()

        @plsc.parallel_loop(0, EB, unroll=2)
        def _(e):
            ps = []
            for h in range(HC):
                a = qb[e, pl.ds(32 * h, 16)] * kb[e, pl.ds(32 * h, 16)]
                b = qb[e, pl.ds(32 * h + 16, 16)] * kb[e, pl.ds(32 * h + 16, 16)]
                ps.append(a + b)
            tots = [jnp.sum(p) for p in ps]
            exvs = [jnp.exp(jnp.broadcast_to(t, (16,))) for t in tots]
            denv = zero16
            for h in range(HC):
                stgv[e, pl.ds(32 * h, 16)] = vb[e, pl.ds(32 * h, 16)] * exvs[h]
                stgv[e, pl.ds(32 * h + 16, 16)] = \
                    vb[e, pl.ds(32 * h + 16, 16)] * exvs[h]
                denv = jnp.where(lane == h, exvs[h], denv)
            exb[lax.shift_right_logical(e, 3),
                pl.ds(jnp.bitwise_and(e, 7) * 16, 16)] = denv

        ce = pltpu.async_copy(exb, exc_hbm.at[c, pl.ds((s * NBLK + i) *
                                                       (EB // 8), EB // 8)],
                              sems)

        @pl.loop(0, EB // 16)
        def _(j):
            rows = jnp.broadcast_to(2 * j, (16,)) + lane8
            erow = jnp.broadcast_to(16 * j, (16,)) + lane
            cv = dstc[pl.ds(16 * j, 16)]
            for h in range(HC):
                exv = plsc.load_gather(exb, [rows, lanec + h])
                plsc.store_scatter(stgd, [erow, cv + h], exv)

        cn = pltpu.async_copy(stgv, num_sh.at[dstn], sems, add=True)
        cd = pltpu.async_copy(stgd, den_sh.at[dstd], sems, add=True)
        ce.wait(); cn.wait(); cd.wait()

        @pl.loop(0, EB // 16)
        def _(j):
            erow = jnp.broadcast_to(16 * j, (16,)) + lane
            cv = dstc[pl.ds(16 * j, 16)]
            for h in range(HC):
                plsc.store_scatter(stgd, [erow, cv + h], zero16)

    plsc.subcore_barrier()
    pltpu.sync_copy(num_sh.at[pl.ds(rbase, RPN)],
                    outn_hbm.at[c, pl.ds(rbase, RPN)])

    @pl.when(s < 4)
    def _():
        dbase = s * (NPD // 4)
        pltpu.sync_copy(den_sh.at[pl.ds(dbase, NPD // 4)],
                        outd_hbm.at[c, pl.ds(dbase, NPD // 4)])
    plsc.subcore_barrier()

    _zero_stgv(stgv)
    _zero_num(stgv, num_sh, s)
    plsc.subcore_barrier()

    @pl.loop(0, NBLK)
    def _(i):
        ebase = (s * NBLK + i) * EB
        pltpu.sync_copy(src_hbm.at[pl.ds(ebase, EB)], srci)
        pltpu.sync_copy(dst_hbm.at[pl.ds(ebase, EB)], dsti)

        @pl.loop(0, EB // 16)
        def _(j):
            sl = pl.ds(16 * j, 16)
            dv = dsti[sl]
            srci[sl] = srci[sl] + cN
            dstn[sl] = jnp.where(dv >= NH, dv - NH, NPH - 1)

        cv = pltpu.async_copy(v_hbm.at[srci], vb, semg)
        ce = pltpu.async_copy(exc_hbm.at[c, pl.ds((s * NBLK + i) * (EB // 8),
                                                  EB // 8)], exb, semg)
        cv.wait(); ce.wait()

        @plsc.parallel_loop(0, EB, unroll=4)
        def _(e):
            er = lax.shift_right_logical(e, 3)
            ec = jnp.bitwise_and(e, 7) * 16
            exvs = [plsc.load_gather(
                exb, [jnp.broadcast_to(er, (16,)),
                      jnp.broadcast_to(ec + h, (16,))]) for h in range(HC)]
            for h in range(HC):
                stgv[e, pl.ds(32 * h, 16)] = vb[e, pl.ds(32 * h, 16)] * exvs[h]
                stgv[e, pl.ds(32 * h + 16, 16)] = \
                    vb[e, pl.ds(32 * h + 16, 16)] * exvs[h]

        pltpu.sync_copy(stgv, num_sh.at[dstn], add=True)

    plsc.subcore_barrier()
    pltpu.sync_copy(num_sh.at[pl.ds(rbase, RPN)],
                    outn_hbm.at[c, pl.ds(NH + rbase, RPN)])
    plsc.subcore_barrier()


def _edge_sc_body(q0, k0, v0, src0, dst0, q1, k1, v1, src1, dst1,
                  outn0, outd0, exc0, outn1, outd1, exc1,
                  srci, dsti, dstq, dstd, dstc, dstn, qb, kb, vb, stgv, stgd,
                  exb, num_sh, den_sh, semg, sems):
    c = lax.axis_index("c")
    s = lax.axis_index("s")
    _edge_one_relation(q0, k0, v0, src0, dst0, outn0, outd0, exc0, srci, dsti,
                       dstq, dstd, dstc, dstn, qb, kb, vb, stgv, stgd, exb,
                       num_sh, den_sh, semg, sems, c, s)
    _edge_one_relation(q1, k1, v1, src1, dst1, outn1, outd1, exc1, srci, dsti,
                       dstq, dstd, dstc, dstn, qb, kb, vb, stgv, stgd, exb,
                       num_sh, den_sh, semg, sems, c, s)


_sc_params = pltpu.CompilerParams()
if "needs_layout_passes" in pltpu.CompilerParams.__dataclass_fields__:
    _sc_params = dataclasses.replace(_sc_params, needs_layout_passes=False)

_edge_sc = functools.partial(
    pl.kernel,
    out_type=[
        jax.ShapeDtypeStruct((2, NOUT, DC), jnp.float32),
        jax.ShapeDtypeStruct((2, NPD, DC), jnp.float32),
        jax.ShapeDtypeStruct((2, EP // 8, DC), jnp.float32),
        jax.ShapeDtypeStruct((2, NOUT, DC), jnp.float32),
        jax.ShapeDtypeStruct((2, NPD, DC), jnp.float32),
        jax.ShapeDtypeStruct((2, EP // 8, DC), jnp.float32),
    ],
    mesh=plsc.VectorSubcoreMesh(core_axis_name="c", subcore_axis_name="s"),
    compiler_params=_sc_params,
    scratch_types=[
        pltpu.VMEM((EB,), jnp.int32),
        pltpu.VMEM((EB,), jnp.int32),
        pltpu.VMEM((EB,), jnp.int32),
        pltpu.VMEM((EB,), jnp.int32),
        pltpu.VMEM((EB,), jnp.int32),
        pltpu.VMEM((EB,), jnp.int32),
        pltpu.VMEM((EB, DC), jnp.float32),
        pltpu.VMEM((EB, DC), jnp.float32),
        pltpu.VMEM((EB, DC), jnp.float32),
        pltpu.VMEM((EB, DC), jnp.float32),
        pltpu.VMEM((EB, DC), jnp.float32),
        pltpu.VMEM((EB // 8, DC), jnp.float32),
        pltpu.VMEM_SHARED((NPH, DC), jnp.float32),
        pltpu.VMEM_SHARED((NPD, DC), jnp.float32),
        pltpu.SemaphoreType.DMA,
        pltpu.SemaphoreType.DMA,
    ],
)(_edge_sc_body)


def kernel(nodes__author, nodes__paper, edges__author__writes__paper,
           edges__paper__written_by__author, Wk, Wq, Wv, Wo, bk, bq, bv, bo,
           a_rel, m_rel, p_rel, skip, ln_gamma, ln_beta):
    scale = 1.0 / jnp.sqrt(jnp.float32(DH))
    a_s = a_rel * (p_rel * scale)[..., None, None]

    Wk_h = Wk.reshape(L, T, D, H, DH)
    Wv_h = Wv.reshape(L, T, D, H, DH)
    bk_h = bk.reshape(L, T, H, DH)
    bv_h = bv.reshape(L, T, H, DH)
    Kf = jnp.einsum('lrdhe,lrhef->lrdhf',
                    jnp.stack([Wk_h[:, 0], Wk_h[:, 1]], axis=1), a_s).reshape(L, R, D, D)
    Vf = jnp.einsum('lrdhe,lrhef->lrdhf',
                    jnp.stack([Wv_h[:, 0], Wv_h[:, 1]], axis=1), m_rel).reshape(L, R, D, D)
    bKf = jnp.einsum('lrhe,lrhef->lrhf',
                     jnp.stack([bk_h[:, 0], bk_h[:, 1]], axis=1), a_s).reshape(L, R, D)
    bVf = jnp.einsum('lrhe,lrhef->lrhf',
                     jnp.stack([bv_h[:, 0], bv_h[:, 1]], axis=1), m_rel).reshape(L, R, D)
    Wcat = jnp.concatenate([Wq, Kf, Vf], axis=3)
    bcat = jnp.concatenate([bq, bKf, bVf], axis=2)[:, :, None, :]

    betas = jax.nn.sigmoid(skip)

    def _pad_edges(e):
        src = jnp.concatenate([e[0].astype(jnp.int32),
                               jnp.zeros((EP - E,), jnp.int32)])
        dst = jnp.concatenate([e[1].astype(jnp.int32),
                               jnp.full((EP - E,), N, jnp.int32)])
        return src, dst

    src_r, dst_r = zip(_pad_edges(edges__author__writes__paper),
                       _pad_edges(edges__paper__written_by__author))

    def layer_body(carry, wl):
        x = list(carry)
        Wcat_l, bcat_l, Wo_l, bo_l, betas_l, gamma_l, bln_l = wl
        q = [None, None]; k = [None, None]; v = [None, None]
        for t in range(T):
            qt, kt, vt = _proj(x[t], Wcat_l[t], bcat_l[t])
            q[t] = qt.reshape(2 * N, DC)
            k[t] = kt.reshape(2 * N, DC)
            v[t] = vt.reshape(2 * N, DC)
        numn0, dend0, _exc0, numn1, dend1, _exc1 = _edge_sc(
            q[1], k[0], v[0], src_r[0], dst_r[0],
            q[0], k[1], v[1], src_r[1], dst_r[1])
        nums = [numn0, numn1]
        dens = []
        for r in range(R):
            d8 = [dend0, dend1][r].reshape(2, NPD * 32, 4)[:, :N, :]
            dens.append(jnp.transpose(d8, (1, 0, 2)).reshape(N, H))
        newx = []
        for t in range(T):
            r = 1 - t
            newx.append(_post(nums[r], dens[r], x[t], Wo_l[t],
                              bo_l[t][None, :], betas_l[t][None, None],
                              gamma_l[None, :], bln_l[None, :]))
        return tuple(newx), None

    carry, _ = lax.scan(layer_body, (nodes__author, nodes__paper),
                        (Wcat, bcat, Wo, bo, betas, ln_gamma, ln_beta))
    return jnp.stack(list(carry), 0)


# speedup vs baseline: 20.5497x; 1.0264x over previous
import dataclasses
import functools

import jax
import jax.numpy as jnp
from jax import lax
from jax.experimental import pallas as pl
from jax.experimental.pallas import tpu as pltpu
from jax.experimental.pallas import tpu_sc as plsc

L = 2; T = 2; R = 2; H = 8; D = 256; DH = 32; N = 10000; E = 100000
BN = 1000
NH = 5000
NPH = 5120
NOUT = NH + NPH
NPD = 320
HC = H // 2
DC = HC * DH
NSUB = 16
EB = 128
NBLK = 49
EP = NSUB * EB * NBLK
RPN = NPH // NSUB

_INTERPRET = False


def _proj_body(x_ref, w_ref, b_ref, q_ref, k_ref, v_ref):
    y = jnp.dot(x_ref[...], w_ref[...], preferred_element_type=jnp.float32)
    y = y + b_ref[...]
    q_ref[0] = y[:, 0:DC]
    q_ref[1] = y[:, DC:2 * DC]
    k_ref[0] = y[:, 256:256 + DC]
    k_ref[1] = y[:, 256 + DC:512]
    v_ref[0] = y[:, 512:512 + DC]
    v_ref[1] = y[:, 512 + DC:768]


def _proj(x, wcat, bcat):
    out3 = [
        jax.ShapeDtypeStruct((2, N, DC), jnp.float32),
        jax.ShapeDtypeStruct((2, N, DC), jnp.float32),
        jax.ShapeDtypeStruct((2, N, DC), jnp.float32),
    ]
    return pl.pallas_call(
        _proj_body,
        grid=(N // BN,),
        in_specs=[
            pl.BlockSpec((BN, D), lambda i: (i, 0)),
            pl.BlockSpec((D, 3 * D), lambda i: (0, 0)),
            pl.BlockSpec((1, 3 * D), lambda i: (0, 0)),
        ],
        out_specs=[pl.BlockSpec((2, BN, DC), lambda i: (0, i, 0))] * 3,
        out_shape=out3,
        interpret=_INTERPRET,
    )(x, wcat, bcat)


def _post_body(num_ref, den_ref, x_ref, wo_ref, bo_ref, beta_ref, g_ref,
               b2_ref, o_ref):
    a = jnp.concatenate([num_ref[0], num_ref[1]], axis=1)
    den = den_ref[...]
    dfull = jnp.broadcast_to(den[:, :, None], (BN, H, DH)).reshape(BN, D)
    agg = a / (dfull + 1e-16)
    g = jax.nn.gelu(agg)
    o = jnp.dot(g, wo_ref[...], preferred_element_type=jnp.float32) + bo_ref[...]
    beta = beta_ref[0, 0]
    x = x_ref[...]
    y = x + beta * o + (1.0 - beta) * x
    mu = jnp.mean(y, axis=1, keepdims=True)
    var = jnp.mean((y - mu) ** 2, axis=1, keepdims=True)
    o_ref[...] = (y - mu) * jax.lax.rsqrt(var + 1e-5) * g_ref[...] + b2_ref[...]


def _post(num, den, x, wo, bo, beta, gamma, beta_ln):
    return pl.pallas_call(
        _post_body,
        grid=(N // BN,),
        in_specs=[
            pl.BlockSpec((2, BN, DC), lambda i: (0, i, 0)),
            pl.BlockSpec((BN, H), lambda i: (i, 0)),
            pl.BlockSpec((BN, D), lambda i: (i, 0)),
            pl.BlockSpec((D, D), lambda i: (0, 0)),
            pl.BlockSpec((1, D), lambda i: (0, 0)),
            pl.BlockSpec((1, 1), lambda i: (0, 0), memory_space=pltpu.SMEM),
            pl.BlockSpec((1, D), lambda i: (0, 0)),
            pl.BlockSpec((1, D), lambda i: (0, 0)),
        ],
        out_specs=pl.BlockSpec((BN, D), lambda i: (i, 0)),
        out_shape=jax.ShapeDtypeStruct((N, D), jnp.float32),
        interpret=_INTERPRET,
    )(num, den, x, wo, bo, beta, gamma, beta_ln)


def _zero_num(stgv, num_sh, s):
    rbase = RPN * s
    for i in range(RPN // EB):
        pltpu.sync_copy(stgv.at[pl.ds(0, EB)],
                        num_sh.at[pl.ds(rbase + i * EB, EB)])
    tail = RPN - (RPN // EB) * EB
    if tail:
        pltpu.sync_copy(stgv.at[pl.ds(0, tail)],
                        num_sh.at[pl.ds(rbase + (RPN // EB) * EB, tail)])


def _zero_stgv(stgv):
    zero16 = jnp.zeros((16,), jnp.float32)

    @pl.loop(0, EB)
    def _(row):
        for j in range(DC // 16):
            stgv[row, pl.ds(16 * j, 16)] = zero16


def _edge_one_relation(q_hbm, k_hbm, v_hbm, src_hbm, dst_hbm, outn_hbm,
                       outd_hbm, exc_hbm, srcall, dstall, srci, dstq, dstd,
                       dstc, dstn, qb, kb, vb, stgv, stgd, exb, num_sh, den_sh,
                       semg, sems, c, s):
    zero16 = jnp.zeros((16,), jnp.float32)
    lane = lax.iota(jnp.int32, 16)
    lane8 = lax.shift_right_logical(lane, 3)
    lanec = jnp.bitwise_and(lane, 7) * 16

    _zero_stgv(stgv)

    @pl.loop(0, EB)
    def _(row):
        for j in range(DC // 16):
            stgd[row, pl.ds(16 * j, 16)] = zero16

    _zero_num(stgv, num_sh, s)

    @pl.when(s == 0)
    def _():
        pltpu.sync_copy(stgv.at[pl.ds(0, EB)], den_sh.at[pl.ds(0, EB)])
        pltpu.sync_copy(stgv.at[pl.ds(0, EB)], den_sh.at[pl.ds(EB, EB)])
        pltpu.sync_copy(stgv.at[pl.ds(0, NPD - 2 * EB)],
                        den_sh.at[pl.ds(2 * EB, NPD - 2 * EB)])
    plsc.subcore_barrier()

    cN = c * N
    rbase = RPN * s

    @pl.loop(0, NBLK)
    def _(i):
        ebase = (s * NBLK + i) * EB
        ca = pltpu.async_copy(src_hbm.at[pl.ds(ebase, EB)], srcall, semg)
        cb = pltpu.async_copy(dst_hbm.at[pl.ds(ebase, EB)], dstall, semg)
        ca.wait(); cb.wait()

        @pl.loop(0, EB // 16)
        def _(j):
            sl = pl.ds(16 * j, 16)
            bl = sl
            dv = dstall[bl]
            srci[sl] = srcall[bl] + cN
            dstq[sl] = jnp.minimum(dv, N - 1) + cN
            dstd[sl] = lax.shift_right_logical(dv, 5)
            dstc[sl] = jnp.bitwise_and(dv, 31) * 4
            dstn[sl] = jnp.minimum(dv, NPH - 1)

        cq = pltpu.async_copy(q_hbm.at[dstq], qb, semg)
        ck = pltpu.async_copy(k_hbm.at[srci], kb, semg)
        cv = pltpu.async_copy(v_hbm.at[srci], vb, semg)
        cq.wait(); ck.wait(); cv.wait()

        @plsc.parallel_loop(0, EB, unroll=4)
        def _(e):
            ps = []
            for h in range(HC):
                a = qb[e, pl.ds(32 * h, 16)] * kb[e, pl.ds(32 * h, 16)]
                b = qb[e, pl.ds(32 * h + 16, 16)] * kb[e, pl.ds(32 * h + 16, 16)]
                ps.append(a + b)
            tots = [jnp.sum(p) for p in ps]
            exvs = [jnp.exp(jnp.broadcast_to(t, (16,))) for t in tots]
            denv = zero16
            for h in range(HC):
                stgv[e, pl.ds(32 * h, 16)] = vb[e, pl.ds(32 * h, 16)] * exvs[h]
                stgv[e, pl.ds(32 * h + 16, 16)] = \
                    vb[e, pl.ds(32 * h + 16, 16)] * exvs[h]
                denv = jnp.where(lane == h, exvs[h], denv)
            exb[lax.shift_right_logical(e, 3),
                pl.ds(jnp.bitwise_and(e, 7) * 16, 16)] = denv

        ce = pltpu.async_copy(exb, exc_hbm.at[c, pl.ds((s * NBLK + i) *
                                                       (EB // 8), EB // 8)],
                              sems)

        @pl.loop(0, EB // 16)
        def _(j):
            rows = jnp.broadcast_to(2 * j, (16,)) + lane8
            erow = jnp.broadcast_to(16 * j, (16,)) + lane
            cv = dstc[pl.ds(16 * j, 16)]
            for h in range(HC):
                exv = plsc.load_gather(exb, [rows, lanec + h])
                plsc.store_scatter(stgd, [erow, cv + h], exv)

        cn = pltpu.async_copy(stgv, num_sh.at[dstn], sems, add=True)
        cd = pltpu.async_copy(stgd, den_sh.at[dstd], sems, add=True)
        ce.wait(); cn.wait(); cd.wait()

        @pl.loop(0, EB // 16)
        def _(j):
            erow = jnp.broadcast_to(16 * j, (16,)) + lane
            cv = dstc[pl.ds(16 * j, 16)]
            for h in range(HC):
                plsc.store_scatter(stgd, [erow, cv + h], zero16)

    plsc.subcore_barrier()
    pltpu.sync_copy(num_sh.at[pl.ds(rbase, RPN)],
                    outn_hbm.at[c, pl.ds(rbase, RPN)])

    @pl.when(s < 4)
    def _():
        dbase = s * (NPD // 4)
        pltpu.sync_copy(den_sh.at[pl.ds(dbase, NPD // 4)],
                        outd_hbm.at[c, pl.ds(dbase, NPD // 4)])
    plsc.subcore_barrier()

    _zero_stgv(stgv)
    _zero_num(stgv, num_sh, s)
    plsc.subcore_barrier()

    @pl.loop(0, NBLK)
    def _(i):
        ebase = (s * NBLK + i) * EB
        ca = pltpu.async_copy(src_hbm.at[pl.ds(ebase, EB)], srcall, semg)
        cb = pltpu.async_copy(dst_hbm.at[pl.ds(ebase, EB)], dstall, semg)
        ca.wait(); cb.wait()

        @pl.loop(0, EB // 16)
        def _(j):
            sl = pl.ds(16 * j, 16)
            dv = dstall[sl]
            srci[sl] = srcall[sl] + cN
            dstn[sl] = jnp.where(dv >= NH, dv - NH, NPH - 1)

        cv = pltpu.async_copy(v_hbm.at[srci], vb, semg)
        ce = pltpu.async_copy(exc_hbm.at[c, pl.ds((s * NBLK + i) * (EB // 8),
                                                  EB // 8)], exb, semg)
        cv.wait(); ce.wait()

        @plsc.parallel_loop(0, EB, unroll=4)
        def _(e):
            er = lax.shift_right_logical(e, 3)
            ec = jnp.bitwise_and(e, 7) * 16
            exvs = [plsc.load_gather(
                exb, [jnp.broadcast_to(er, (16,)),
                      jnp.broadcast_to(ec + h, (16,))]) for h in range(HC)]
            for h in range(HC):
                stgv[e, pl.ds(32 * h, 16)] = vb[e, pl.ds(32 * h, 16)] * exvs[h]
                stgv[e, pl.ds(32 * h + 16, 16)] = \
                    vb[e, pl.ds(32 * h + 16, 16)] * exvs[h]

        pltpu.sync_copy(stgv, num_sh.at[dstn], add=True)

    plsc.subcore_barrier()
    pltpu.sync_copy(num_sh.at[pl.ds(rbase, RPN)],
                    outn_hbm.at[c, pl.ds(NH + rbase, RPN)])
    plsc.subcore_barrier()


def _edge_sc_body(q0, k0, v0, src0, dst0, q1, k1, v1, src1, dst1,
                  outn0, outd0, exc0, outn1, outd1, exc1,
                  srcall, dstall, srci, dstq, dstd, dstc, dstn, qb, kb, vb,
                  stgv, stgd, exb, num_sh, den_sh, semg, sems):
    c = lax.axis_index("c")
    s = lax.axis_index("s")
    _edge_one_relation(q0, k0, v0, src0, dst0, outn0, outd0, exc0, srcall,
                       dstall, srci, dstq, dstd, dstc, dstn, qb, kb, vb, stgv,
                       stgd, exb, num_sh, den_sh, semg, sems, c, s)
    _edge_one_relation(q1, k1, v1, src1, dst1, outn1, outd1, exc1, srcall,
                       dstall, srci, dstq, dstd, dstc, dstn, qb, kb, vb, stgv,
                       stgd, exb, num_sh, den_sh, semg, sems, c, s)


_sc_params = pltpu.CompilerParams()
if "needs_layout_passes" in pltpu.CompilerParams.__dataclass_fields__:
    _sc_params = dataclasses.replace(_sc_params, needs_layout_passes=False)

_edge_sc = functools.partial(
    pl.kernel,
    out_type=[
        jax.ShapeDtypeStruct((2, NOUT, DC), jnp.float32),
        jax.ShapeDtypeStruct((2, NPD, DC), jnp.float32),
        jax.ShapeDtypeStruct((2, EP // 8, DC), jnp.float32),
        jax.ShapeDtypeStruct((2, NOUT, DC), jnp.float32),
        jax.ShapeDtypeStruct((2, NPD, DC), jnp.float32),
        jax.ShapeDtypeStruct((2, EP // 8, DC), jnp.float32),
    ],
    mesh=plsc.VectorSubcoreMesh(core_axis_name="c", subcore_axis_name="s"),
    compiler_params=_sc_params,
    scratch_types=[
        pltpu.VMEM((EB,), jnp.int32),
        pltpu.VMEM((EB,), jnp.int32),
        pltpu.VMEM((EB,), jnp.int32),
        pltpu.VMEM((EB,), jnp.int32),
        pltpu.VMEM((EB,), jnp.int32),
        pltpu.VMEM((EB,), jnp.int32),
        pltpu.VMEM((EB,), jnp.int32),
        pltpu.VMEM((EB, DC), jnp.float32),
        pltpu.VMEM((EB, DC), jnp.float32),
        pltpu.VMEM((EB, DC), jnp.float32),
        pltpu.VMEM((EB, DC), jnp.float32),
        pltpu.VMEM((EB, DC), jnp.float32),
        pltpu.VMEM((EB // 8, DC), jnp.float32),
        pltpu.VMEM_SHARED((NPH, DC), jnp.float32),
        pltpu.VMEM_SHARED((NPD, DC), jnp.float32),
        pltpu.SemaphoreType.DMA,
        pltpu.SemaphoreType.DMA,
    ],
)(_edge_sc_body)


def kernel(nodes__author, nodes__paper, edges__author__writes__paper,
           edges__paper__written_by__author, Wk, Wq, Wv, Wo, bk, bq, bv, bo,
           a_rel, m_rel, p_rel, skip, ln_gamma, ln_beta):
    scale = 1.0 / jnp.sqrt(jnp.float32(DH))
    a_s = a_rel * (p_rel * scale)[..., None, None]

    Wk_h = Wk.reshape(L, T, D, H, DH)
    Wv_h = Wv.reshape(L, T, D, H, DH)
    bk_h = bk.reshape(L, T, H, DH)
    bv_h = bv.reshape(L, T, H, DH)
    Kf = jnp.einsum('lrdhe,lrhef->lrdhf',
                    jnp.stack([Wk_h[:, 0], Wk_h[:, 1]], axis=1), a_s).reshape(L, R, D, D)
    Vf = jnp.einsum('lrdhe,lrhef->lrdhf',
                    jnp.stack([Wv_h[:, 0], Wv_h[:, 1]], axis=1), m_rel).reshape(L, R, D, D)
    bKf = jnp.einsum('lrhe,lrhef->lrhf',
                     jnp.stack([bk_h[:, 0], bk_h[:, 1]], axis=1), a_s).reshape(L, R, D)
    bVf = jnp.einsum('lrhe,lrhef->lrhf',
                     jnp.stack([bv_h[:, 0], bv_h[:, 1]], axis=1), m_rel).reshape(L, R, D)
    Wcat = jnp.concatenate([Wq, Kf, Vf], axis=3)
    bcat = jnp.concatenate([bq, bKf, bVf], axis=2)[:, :, None, :]

    betas = jax.nn.sigmoid(skip)

    def _pad_edges(e):
        src = jnp.concatenate([e[0].astype(jnp.int32),
                               jnp.zeros((EP - E,), jnp.int32)])
        dst = jnp.concatenate([e[1].astype(jnp.int32),
                               jnp.full((EP - E,), N, jnp.int32)])
        return src, dst

    src_r, dst_r = zip(_pad_edges(edges__author__writes__paper),
                       _pad_edges(edges__paper__written_by__author))

    def layer_body(carry, wl):
        x = list(carry)
        Wcat_l, bcat_l, Wo_l, bo_l, betas_l, gamma_l, bln_l = wl
        q = [None, None]; k = [None, None]; v = [None, None]
        for t in range(T):
            qt, kt, vt = _proj(x[t], Wcat_l[t], bcat_l[t])
            q[t] = qt.reshape(2 * N, DC)
            k[t] = kt.reshape(2 * N, DC)
            v[t] = vt.reshape(2 * N, DC)
        numn0, dend0, _exc0, numn1, dend1, _exc1 = _edge_sc(
            q[1], k[0], v[0], src_r[0], dst_r[0],
            q[0], k[1], v[1], src_r[1], dst_r[1])
        nums = [numn0, numn1]
        dens = []
        for r in range(R):
            d8 = [dend0, dend1][r].reshape(2, NPD * 32, 4)[:, :N, :]
            dens.append(jnp.transpose(d8, (1, 0, 2)).reshape(N, H))
        newx = []
        for t in range(T):
            r = 1 - t
            newx.append(_post(nums[r], dens[r], x[t], Wo_l[t],
                              bo_l[t][None, :], betas_l[t][None, None],
                              gamma_l[None, :], bln_l[None, :]))
        return tuple(newx), None

    carry, _ = lax.scan(layer_body, (nodes__author, nodes__paper),
                        (Wcat, bcat, Wo, bo, betas, ln_gamma, ln_beta))
    return jnp.stack(list(carry), 0)
